# baseline (speedup 1.0000x reference)
import jax
import jax.numpy as jnp
from jax import lax
from jax.experimental import pallas as pl
from jax.experimental.pallas import tpu as pltpu
from jax.experimental.pallas import tpu_sc as plsc

NUM_CORES = 2
NUM_SUBCORES = 16
LANES = 16
NUM_WORKERS = NUM_CORES * NUM_SUBCORES

BATCH = 16384
EMBED_DIM = 32
TILE_W = 128
B_PER_W = BATCH // NUM_WORKERS
CHUNK = 16
NUM_CHUNKS = B_PER_W // CHUNK


def _window_prologue(ids, lanes16, shift_v):
    col = (ids >> 7) << 7
    shift_v[...] = col
    prev = plsc.load_gather(shift_v, [jnp.maximum(lanes16 - 1, 0)])
    is_new = jnp.logical_or(col != prev, lanes16 == 0).astype(jnp.int32)
    slots = plsc.cumsum(is_new) - 1
    col_s = [jnp.sum(jnp.where(lanes16 == q, col, 0)) for q in range(CHUNK)]
    return col, col_s, slots


def _fetch_dedup(tbl_hbm, col_s, wbuf, sem):
    cnt = jnp.int32(0)
    for q in range(CHUNK):
        new_q = (col_s[q] != col_s[q - 1]) if q else None
        def fire(cq=col_s[q], c=cnt):
            pltpu.async_copy(
                tbl_hbm.at[:, pl.ds(pl.multiple_of(cq, TILE_W), TILE_W)],
                wbuf.at[pl.ds(c * EMBED_DIM, EMBED_DIM), :], sem)
        if q == 0:
            fire()
            cnt = cnt + 1
        else:
            pl.when(new_q)(fire)
            cnt = cnt + new_q.astype(jnp.int32)
    return cnt


def _drain(tbl_hbm, wbuf, sem, cnt):
    def body(_, carry):
        pltpu.make_async_copy(
            tbl_hbm.at[:, pl.ds(0, TILE_W)],
            wbuf.at[pl.ds(0, EMBED_DIM), :], sem).wait()
        return carry
    lax.fori_loop(0, cnt, body, 0)


def _gather_kernel(sids_hbm, tbl_hbm, out_hbm, idx_v, wbuf, obuf, shift_v,
                   sem):
    wid = lax.axis_index("s") * NUM_CORES + lax.axis_index("c")
    base = wid * B_PER_W
    pltpu.sync_copy(sids_hbm.at[pl.ds(base, B_PER_W)], idx_v)
    lanes16 = lax.iota(jnp.int32, LANES)

    def chunk_body(cb, carry):
        c0 = cb * CHUNK
        ids = idx_v[pl.ds(c0, CHUNK)]
        col, col_s, slots = _window_prologue(ids, lanes16, shift_v)
        cnt = _fetch_dedup(tbl_hbm, col_s, wbuf, sem)
        _drain(tbl_hbm, wbuf, sem, cnt)
        lane = ids & (TILE_W - 1)
        row = slots * EMBED_DIM
        for d in range(EMBED_DIM):
            vals = plsc.load_gather(wbuf, [row + d, lane])
            plsc.store_scatter(obuf, [lanes16, lanes16 * 0 + d], vals)
        pltpu.sync_copy(obuf, out_hbm.at[pl.ds(base + c0, CHUNK), :])
        return carry

    lax.fori_loop(0, NUM_CHUNKS, chunk_body, 0)


def _score_kernel(sids_hbm, jpos_hbm, tbl_hbm, uv_hbm, out_hbm, idx_v, jv,
                  wbuf, ubuf, out_v, shift_v, sem, sem_u):
    wid = lax.axis_index("s") * NUM_CORES + lax.axis_index("c")
    base = wid * B_PER_W
    pltpu.sync_copy(sids_hbm.at[pl.ds(base, B_PER_W)], idx_v)
    pltpu.sync_copy(jpos_hbm.at[pl.ds(base, B_PER_W)], jv)
    lanes16 = lax.iota(jnp.int32, LANES)

    def chunk_body(cb, carry):
        c0 = cb * CHUNK
        jvec = jv[pl.ds(c0, CHUNK)]
        ucp = pltpu.async_copy(uv_hbm.at[jvec], ubuf, sem_u)
        ids = idx_v[pl.ds(c0, CHUNK)]
        col, col_s, slots = _window_prologue(ids, lanes16, shift_v)
        cnt = _fetch_dedup(tbl_hbm, col_s, wbuf, sem)
        _drain(tbl_hbm, wbuf, sem, cnt)
        ucp.wait()
        lane = ids & (TILE_W - 1)
        row = slots * EMBED_DIM
        acc = jnp.zeros((LANES,), jnp.float32)
        for d in range(EMBED_DIM):
            gi = plsc.load_gather(wbuf, [row + d, lane])
            gu = plsc.load_gather(ubuf, [lanes16, lanes16 * 0 + d])
            acc = acc + gu * gi
        out_v[pl.ds(c0, CHUNK)] = acc
        return carry

    lax.fori_loop(0, NUM_CHUNKS, chunk_body, 0)

    pltpu.sync_copy(out_v, out_hbm.at[pl.ds(base, B_PER_W)])


def _mesh():
    return plsc.VectorSubcoreMesh(
        core_axis_name="c", subcore_axis_name="s",
        num_cores=NUM_CORES, num_subcores=NUM_SUBCORES)


@jax.jit
def kernel(user_ids, item_ids, user_table, item_table):
    iota = jnp.arange(BATCH, dtype=jnp.int32)
    su, pu = lax.sort_key_val(user_ids.astype(jnp.int32), iota)
    si, pi = lax.sort_key_val(item_ids.astype(jnp.int32), iota)
    rank_u = jnp.zeros((BATCH,), jnp.int32).at[pu].set(iota)
    jpos = rank_u[pi]
    inv_i = jnp.zeros((BATCH,), jnp.int32).at[pi].set(iota)

    gather_run = pl.kernel(
        _gather_kernel,
        out_type=jax.ShapeDtypeStruct((BATCH, TILE_W), jnp.float32),
        mesh=_mesh(),
        scratch_types=[
            pltpu.VMEM((B_PER_W,), jnp.int32),
            pltpu.VMEM((CHUNK * EMBED_DIM, TILE_W), jnp.float32),
            pltpu.VMEM((CHUNK, TILE_W), jnp.float32),
            pltpu.VMEM((LANES,), jnp.int32),
            pltpu.SemaphoreType.DMA,
        ],
        compiler_params=pltpu.CompilerParams(needs_layout_passes=False),
    )
    uv2 = gather_run(su, user_table.T)

    score_run = pl.kernel(
        _score_kernel,
        out_type=jax.ShapeDtypeStruct((BATCH,), jnp.float32),
        mesh=_mesh(),
        scratch_types=[
            pltpu.VMEM((B_PER_W,), jnp.int32),
            pltpu.VMEM((B_PER_W,), jnp.int32),
            pltpu.VMEM((CHUNK * EMBED_DIM, TILE_W), jnp.float32),
            pltpu.VMEM((CHUNK, TILE_W), jnp.float32),
            pltpu.VMEM((B_PER_W,), jnp.float32),
            pltpu.VMEM((LANES,), jnp.int32),
            pltpu.SemaphoreType.DMA,
            pltpu.SemaphoreType.DMA,
        ],
        compiler_params=pltpu.CompilerParams(needs_layout_passes=False),
    )
    scores_sorted = score_run(si, jpos, item_table.T, uv2)
    return scores_sorted[inv_i]
